# Initial kernel scaffold; baseline (speedup 1.0000x reference)
import jax
import jax.numpy as jnp
from jax.experimental import pallas as pl


def kernel(hidden_states, gate_w, w1, w3, w2, w13_bias, w2_bias):
    raise NotImplementedError("write your pallas kernel here")


# speedup vs baseline: 1.4901x; 1.4901x over previous
import functools

import jax
import jax.numpy as jnp
from jax.experimental import pallas as pl
from jax.experimental.pallas import tpu as pltpu

NUM_EXPERTS_C = 8
TOP_K_C = 2
BT = 256


def _router_body(x_ref, g_ref, id0_ref, id1_ref, w0_ref, w1_ref):
    x = x_ref[...]
    g = g_ref[...]
    logits = jax.lax.dot_general(
        x, g, (((1,), (1,)), ((), ())), preferred_element_type=jnp.float32)
    rb, e = logits.shape
    iota = jax.lax.broadcasted_iota(jnp.int32, (rb, e), 1)
    m0 = jnp.max(logits, axis=-1, keepdims=True)
    am0 = jnp.min(jnp.where(logits == m0, iota, e), axis=-1, keepdims=True)
    l2 = jnp.where(iota == am0, -jnp.inf, logits)
    m1 = jnp.max(l2, axis=-1, keepdims=True)
    am1 = jnp.min(jnp.where(l2 == m1, iota, e), axis=-1, keepdims=True)
    t = jnp.exp(m1 - m0)
    w0 = 1.0 / (1.0 + t)
    id0_ref[...] = am0
    id1_ref[...] = am1
    w0_ref[...] = w0
    w1_ref[...] = t * w0


def _run_router(hidden_states, gate_w):
    T, H = hidden_states.shape
    E = gate_w.shape[0]
    RB = 1024
    grid = (T // RB,)
    out_shapes = (
        jax.ShapeDtypeStruct((T, 1), jnp.int32),
        jax.ShapeDtypeStruct((T, 1), jnp.int32),
        jax.ShapeDtypeStruct((T, 1), jnp.float32),
        jax.ShapeDtypeStruct((T, 1), jnp.float32),
    )
    o_spec = pl.BlockSpec((RB, 1), lambda i: (i, 0))
    return pl.pallas_call(
        _router_body,
        grid=grid,
        in_specs=[
            pl.BlockSpec((RB, H), lambda i: (i, 0)),
            pl.BlockSpec((E, H), lambda i: (0, 0)),
        ],
        out_specs=(o_spec, o_spec, o_spec, o_spec),
        out_shape=out_shapes,
    )(hidden_states, gate_w)


def _bookkeeping(id0, id1, w0, w1, T, E, NBmax):
    eflat = jnp.concatenate([id0[:, 0], id1[:, 0]])
    wflat = jnp.concatenate([w0[:, 0], w1[:, 0]])
    A = eflat.shape[0]
    oh = (eflat[:, None] == jnp.arange(E, dtype=jnp.int32)[None, :]).astype(jnp.int32)
    counts = jnp.sum(oh, axis=0)
    rank = jnp.take_along_axis(jnp.cumsum(oh, axis=0), eflat[:, None], axis=1)[:, 0] - 1
    padded = ((counts + BT - 1) // BT) * BT
    pstart = jnp.concatenate([jnp.zeros((1,), jnp.int32),
                              jnp.cumsum(padded)[:-1].astype(jnp.int32)])
    pos = pstart[eflat] + rank
    nbe = padded // BT
    bstart = jnp.cumsum(nbe).astype(jnp.int32)
    nb = bstart[-1]
    blk = jnp.minimum(jnp.arange(NBmax, dtype=jnp.int32), nb - 1)
    bexp = jnp.searchsorted(bstart, blk, side='right').astype(jnp.int32)
    S = NBmax * BT
    tok = jnp.concatenate([jnp.arange(T, dtype=jnp.int32)] * 2)
    tok_slot = jnp.zeros((S,), jnp.int32).at[pos].set(tok)
    w_slot = jnp.zeros((S,), jnp.float32).at[pos].set(wflat)
    return bexp, nb, tok_slot, w_slot, pos[:T], pos[T:]


def _mlp_body(bexp_ref, nb_ref, x_ref, w1_ref, w3_ref, w2_ref,
              b13_ref, b2_ref, ws_ref, y_ref):
    I = w1_ref.shape[1]

    @pl.when(pl.program_id(0) < nb_ref[0])
    def _():
        x = x_ref[0]
        a = jax.lax.dot_general(
            x, w1_ref[0], (((1,), (1,)), ((), ())),
            preferred_element_type=jnp.float32) + b13_ref[0, :, :I]
        c = jax.lax.dot_general(
            x, w3_ref[0], (((1,), (1,)), ((), ())),
            preferred_element_type=jnp.float32) + b13_ref[0, :, I:]
        h = a * jax.lax.logistic(a) * c
        acc = jax.lax.dot_general(
            h, w2_ref[0], (((1,), (1,)), ((), ())),
            preferred_element_type=jnp.float32)
        y_ref[0] = (acc + b2_ref[0]) * ws_ref[0]


def _run_mlp(x_sorted, w1, w3, w2, w13_bias, w2_bias, w_slot, bexp, nb, NBmax):
    E, I, H = w1.shape
    S = NBmax * BT
    x3 = x_sorted.reshape(NBmax, BT, H)
    ws3 = w_slot.reshape(NBmax, BT, 1)
    nb_arr = jnp.reshape(nb, (1,)).astype(jnp.int32)

    def live(b, bexp_r, nb_r):
        return jnp.minimum(b, nb_r[0] - 1)

    def xmap(b, bexp_r, nb_r):
        return (live(b, bexp_r, nb_r), 0, 0)

    def wmap(b, bexp_r, nb_r):
        return (bexp_r[live(b, bexp_r, nb_r)], 0, 0)

    def bmap3(b, bexp_r, nb_r):
        return (bexp_r[live(b, bexp_r, nb_r)], 0, 0)

    grid_spec = pltpu.PrefetchScalarGridSpec(
        num_scalar_prefetch=2,
        grid=(NBmax,),
        in_specs=[
            pl.BlockSpec((1, BT, H), xmap),
            pl.BlockSpec((1, I, H), wmap),
            pl.BlockSpec((1, I, H), wmap),
            pl.BlockSpec((1, H, I), wmap),
            pl.BlockSpec((1, 1, 2 * I), bmap3),
            pl.BlockSpec((1, 1, H), bmap3),
            pl.BlockSpec((1, BT, 1), xmap),
        ],
        out_specs=pl.BlockSpec((1, BT, H), xmap),
    )
    y3 = pl.pallas_call(
        _mlp_body,
        grid_spec=grid_spec,
        out_shape=jax.ShapeDtypeStruct((NBmax, BT, H), jnp.float32),
        compiler_params=pltpu.CompilerParams(
            dimension_semantics=("arbitrary",),
            vmem_limit_bytes=120 * 1024 * 1024,
        ),
    )(bexp, nb_arr, x3, w1, w3, w2,
      w13_bias.reshape(E, 1, 2 * I), w2_bias.reshape(E, 1, H), ws3)
    return y3.reshape(S, H)


def kernel(hidden_states, gate_w, w1, w3, w2, w13_bias, w2_bias):
    T, H = hidden_states.shape
    E = w1.shape[0]
    A = T * TOP_K_C
    NBmax = A // BT + (E - 1)

    id0, id1, w0, w1r, = _run_router(hidden_states, gate_w)
    bexp, nb, tok_slot, w_slot, pos0, pos1 = _bookkeeping(
        id0, id1, w0, w1r, T, E, NBmax)

    x_sorted = jnp.take(hidden_states, tok_slot, axis=0)
    y = _run_mlp(x_sorted, w1, w3, w2, w13_bias, w2_bias,
                 w_slot, bexp, nb, NBmax)
    return jnp.take(y, pos0, axis=0) + jnp.take(y, pos1, axis=0)
